# Initial kernel scaffold; baseline (speedup 1.0000x reference)
import jax
import jax.numpy as jnp
from jax.experimental import pallas as pl


def kernel(outputs, targets):
    raise NotImplementedError("write your pallas kernel here")


# speedup vs baseline: 1.1604x; 1.1604x over previous
import functools

import jax
import jax.numpy as jnp
from jax import lax
from jax.experimental import pallas as pl
from jax.experimental.pallas import tpu as pltpu
from jax.experimental.pallas import tpu_sc as plsc

B = 1024
N = 100000

NC, NS, L = 2, 16, 16
NW = NC * NS
BPW = B // NW
NROWS16 = (B * N) // L


def _sc_gather_body(xflat_hbm, tgt_hbm, v_hbm, tgt_v, idx_v, val_v, sem):
    wid = lax.axis_index("s") * NC + lax.axis_index("c")
    base = wid * BPW
    pltpu.sync_copy(tgt_hbm.at[pl.ds(base, BPW)], tgt_v)
    for c in range(BPW // L):
        t = tgt_v[pl.ds(c * L, L)]
        row_id = base + c * L + lax.broadcasted_iota(jnp.int32, (L,), 0)
        idx_v[pl.ds(c * L, L)] = row_id * N + t
    pltpu.async_copy(xflat_hbm.at[idx_v], val_v, sem).wait()
    pltpu.sync_copy(val_v, v_hbm.at[pl.ds(base, BPW)])


def _make_sc_gather():
    return functools.partial(
        pl.kernel,
        mesh=plsc.VectorSubcoreMesh(core_axis_name="c", subcore_axis_name="s"),
        out_type=jax.ShapeDtypeStruct((B,), jnp.float32),
        scratch_types=[
            pltpu.VMEM((BPW,), jnp.int32),
            pltpu.VMEM((BPW,), jnp.int32),
            pltpu.VMEM((BPW,), jnp.float32),
            pltpu.SemaphoreType.DMA,
        ],
    )(_sc_gather_body)


RB = 256
CB = 5888
NCB = -(-N // CB)
SCALE = 100.0 / B


def _count_body(x_ref, v_ref, t_ref, c1_ref, c5_ref, acc_ref):
    j = pl.program_id(1)

    @pl.when(j == 0)
    def _init():
        acc_ref[...] = jnp.zeros_like(acc_ref)

    x = x_ref[...]
    v = v_ref[...]
    t = t_ref[...]
    col = j * CB + lax.broadcasted_iota(jnp.int32, (RB, CB), 1)
    beats = (x > v) | ((x == v) & (col < t))
    hit = jnp.where(beats & (col < N), 1, 0)
    acc_ref[...] += jnp.sum(hit, axis=1, keepdims=True)

    @pl.when(j == NCB - 1)
    def _finish():
        rank = acc_ref[...]
        c1 = jnp.sum(jnp.where(rank < 1, SCALE, 0.0))
        c5 = jnp.sum(jnp.where(rank < 5, SCALE, 0.0))

        @pl.when(pl.program_id(0) == 0)
        def _zero():
            c1_ref[0, 0] = 0.0
            c5_ref[0, 0] = 0.0

        c1_ref[0, 0] += c1
        c5_ref[0, 0] += c5


_count = pl.pallas_call(
    _count_body,
    grid=(B // RB, NCB),
    in_specs=[
        pl.BlockSpec((RB, CB), lambda i, j: (i, j)),
        pl.BlockSpec((RB, 1), lambda i, j: (i, 0)),
        pl.BlockSpec((RB, 1), lambda i, j: (i, 0)),
    ],
    out_specs=[
        pl.BlockSpec(memory_space=pltpu.SMEM),
        pl.BlockSpec(memory_space=pltpu.SMEM),
    ],
    out_shape=[jax.ShapeDtypeStruct((1, 1), jnp.float32)] * 2,
    scratch_shapes=[pltpu.VMEM((RB, 1), jnp.int32)],
)


def kernel(outputs, targets):
    tgt = targets.astype(jnp.int32)
    xflat = outputs.reshape(B * N)
    v = _make_sc_gather()(xflat, tgt)
    c1, c5 = _count(outputs, v.reshape(B, 1), tgt.reshape(B, 1))
    return (c1.reshape(1), c5.reshape(1))


# speedup vs baseline: 1.1653x; 1.0042x over previous
import functools

import jax
import jax.numpy as jnp
from jax import lax
from jax.experimental import pallas as pl
from jax.experimental.pallas import tpu as pltpu
from jax.experimental.pallas import tpu_sc as plsc

B = 1024
N = 100000

NC, NS, L = 2, 16, 16
NW = NC * NS
BPW = B // NW
NROWS16 = (B * N) // L


def _sc_gather_body(xflat_hbm, tgt_hbm, v_hbm, tgt_v, idx_v, val_v, sem):
    wid = lax.axis_index("s") * NC + lax.axis_index("c")
    base = wid * BPW
    pltpu.sync_copy(tgt_hbm.at[pl.ds(base, BPW)], tgt_v)
    for c in range(BPW // L):
        t = tgt_v[pl.ds(c * L, L)]
        row_id = base + c * L + lax.broadcasted_iota(jnp.int32, (L,), 0)
        idx_v[pl.ds(c * L, L)] = row_id * N + t
    pltpu.async_copy(xflat_hbm.at[idx_v], val_v, sem).wait()
    pltpu.sync_copy(val_v, v_hbm.at[pl.ds(base, BPW)])


def _make_sc_gather():
    return functools.partial(
        pl.kernel,
        mesh=plsc.VectorSubcoreMesh(core_axis_name="c", subcore_axis_name="s"),
        out_type=jax.ShapeDtypeStruct((B,), jnp.float32),
        scratch_types=[
            pltpu.VMEM((BPW,), jnp.int32),
            pltpu.VMEM((BPW,), jnp.int32),
            pltpu.VMEM((BPW,), jnp.float32),
            pltpu.SemaphoreType.DMA,
        ],
    )(_sc_gather_body)


RB = 256
CB = 5888
NCB = -(-N // CB)
SCALE = 100.0 / B


def _count_body(x_ref, v_ref, t_ref, rank_ref):
    j = pl.program_id(1)

    @pl.when(j == 0)
    def _init():
        rank_ref[...] = jnp.zeros_like(rank_ref)

    x = x_ref[...]
    v = v_ref[...]
    t = t_ref[...]
    col = j * CB + lax.broadcasted_iota(jnp.int32, (RB, CB), 1)
    beats = (x > v) | ((x == v) & (col < t))

    @pl.when(j < NCB - 1)
    def _mid():
        rank_ref[...] += jnp.sum(jnp.where(beats, 1, 0), axis=1, keepdims=True)

    @pl.when(j == NCB - 1)
    def _last():
        hit = jnp.where(beats & (col < N), 1, 0)
        rank_ref[...] += jnp.sum(hit, axis=1, keepdims=True)


_count = pl.pallas_call(
    _count_body,
    grid=(B // RB, NCB),
    in_specs=[
        pl.BlockSpec((RB, CB), lambda i, j: (i, j)),
        pl.BlockSpec((RB, 1), lambda i, j: (i, 0)),
        pl.BlockSpec((RB, 1), lambda i, j: (i, 0)),
    ],
    out_specs=pl.BlockSpec((RB, 1), lambda i, j: (i, 0)),
    out_shape=jax.ShapeDtypeStruct((B, 1), jnp.int32),
    compiler_params=pltpu.CompilerParams(
        dimension_semantics=("parallel", "arbitrary")),
)


def _final_body(rank_ref, c1_ref, c5_ref):
    rank = rank_ref[...]
    c1_ref[0, 0] = jnp.sum(jnp.where(rank < 1, SCALE, 0.0))
    c5_ref[0, 0] = jnp.sum(jnp.where(rank < 5, SCALE, 0.0))


_final = pl.pallas_call(
    _final_body,
    in_specs=[pl.BlockSpec((B, 1), lambda: (0, 0))],
    out_specs=[
        pl.BlockSpec(memory_space=pltpu.SMEM),
        pl.BlockSpec(memory_space=pltpu.SMEM),
    ],
    out_shape=[jax.ShapeDtypeStruct((1, 1), jnp.float32)] * 2,
)


def kernel(outputs, targets):
    tgt = targets.astype(jnp.int32)
    xflat = outputs.reshape(B * N)
    v = _make_sc_gather()(xflat, tgt)
    rank = _count(outputs, v.reshape(B, 1), tgt.reshape(B, 1))
    c1, c5 = _final(rank)
    return (c1.reshape(1), c5.reshape(1))


# speedup vs baseline: 1.2206x; 1.0475x over previous
import functools

import jax
import jax.numpy as jnp
from jax import lax
from jax.experimental import pallas as pl
from jax.experimental.pallas import tpu as pltpu
from jax.experimental.pallas import tpu_sc as plsc

B = 1024
N = 100000

NC, NS, L = 2, 16, 16
NW = NC * NS
BPW = B // NW
NROWS16 = (B * N) // L


def _sc_gather_body(xflat_hbm, tgt_hbm, v_hbm, tgt_v, idx_v, val_v, sem):
    wid = lax.axis_index("s") * NC + lax.axis_index("c")
    base = wid * BPW
    pltpu.sync_copy(tgt_hbm.at[pl.ds(base, BPW)], tgt_v)
    for c in range(BPW // L):
        t = tgt_v[pl.ds(c * L, L)]
        row_id = base + c * L + lax.broadcasted_iota(jnp.int32, (L,), 0)
        idx_v[pl.ds(c * L, L)] = row_id * N + t
    pltpu.async_copy(xflat_hbm.at[idx_v], val_v, sem).wait()
    pltpu.sync_copy(val_v, v_hbm.at[pl.ds(base, BPW)])


def _make_sc_gather():
    return functools.partial(
        pl.kernel,
        mesh=plsc.VectorSubcoreMesh(core_axis_name="c", subcore_axis_name="s"),
        out_type=jax.ShapeDtypeStruct((B,), jnp.float32),
        scratch_types=[
            pltpu.VMEM((BPW,), jnp.int32),
            pltpu.VMEM((BPW,), jnp.int32),
            pltpu.VMEM((BPW,), jnp.float32),
            pltpu.SemaphoreType.DMA,
        ],
    )(_sc_gather_body)


CH = 8
NBUF = 8
NCHUNK = B // CH
GROUPS = NCHUNK // NBUF
SCALE = 100.0 / B


def _stream_body(x_hbm, v_ref, t_ref, c1_ref, c5_ref, rank_v, *bufs_sems):
    bufs = bufs_sems[:NBUF]
    sems = bufs_sems[NBUF:]

    def dma(b, c):
        return pltpu.make_async_copy(
            x_hbm.at[pl.ds(c * CH, CH), :], bufs[b], sems[b])

    for b in range(NBUF):
        dma(b, b).start()

    def group(g, carry):
        for b in range(NBUF):
            c = g * NBUF + b
            dma(b, c).wait()
            x = bufs[b][...]
            base = c * CH
            v = v_ref[pl.ds(base, CH), :]
            t = t_ref[pl.ds(base, CH), :]
            col = lax.broadcasted_iota(jnp.int32, (CH, N), 1)
            beats = (x > v) | ((x == v) & (col < t))
            rank_v[pl.ds(base, CH), :] = jnp.sum(
                jnp.where(beats, 1, 0), axis=1, keepdims=True)

            @pl.when(g < GROUPS - 1)
            def _next():
                dma(b, (g + 1) * NBUF + b).start()
        return carry

    lax.fori_loop(0, GROUPS, group, 0)
    rank = rank_v[...]
    c1_ref[0, 0] = jnp.sum(jnp.where(rank < 1, SCALE, 0.0))
    c5_ref[0, 0] = jnp.sum(jnp.where(rank < 5, SCALE, 0.0))


_stream = pl.pallas_call(
    _stream_body,
    in_specs=[
        pl.BlockSpec(memory_space=pl.ANY),
        pl.BlockSpec(memory_space=pltpu.VMEM),
        pl.BlockSpec(memory_space=pltpu.VMEM),
    ],
    out_specs=[
        pl.BlockSpec(memory_space=pltpu.SMEM),
        pl.BlockSpec(memory_space=pltpu.SMEM),
    ],
    out_shape=[jax.ShapeDtypeStruct((1, 1), jnp.float32)] * 2,
    scratch_shapes=([pltpu.VMEM((B, 1), jnp.int32)]
                    + [pltpu.VMEM((CH, N), jnp.float32)] * NBUF
                    + [pltpu.SemaphoreType.DMA] * NBUF),
)


def kernel(outputs, targets):
    tgt = targets.astype(jnp.int32)
    xflat = outputs.reshape(B * N)
    v = _make_sc_gather()(xflat, tgt)
    c1, c5 = _stream(outputs, v.reshape(B, 1), tgt.reshape(B, 1))
    return (c1.reshape(1), c5.reshape(1))
